# Initial kernel scaffold; baseline (speedup 1.0000x reference)
import jax
import jax.numpy as jnp
from jax.experimental import pallas as pl


def kernel(values, indicators, table):
    raise NotImplementedError("write your pallas kernel here")


# speedup vs baseline: 4.3858x; 4.3858x over previous
import functools

import jax
import jax.numpy as jnp
from jax import lax
from jax.experimental import pallas as pl
from jax.experimental.pallas import tpu as pltpu
from jax.experimental.pallas import tpu_sc as plsc

_N_QUANTILES = 1000
_NUM_TOKENS = 3
_D = 64

_NC = 2
_NS = 16
_NW = _NC * _NS
_LANES = 16

_CH = 1024
_GCH = 128


def _embed_body(vals_hbm, ind_hbm, table_hbm, out_hbm,
                vals_v, ind_v, idx_v, rows_v, sem):
    b = vals_hbm.shape[0]
    tpw = b // _NW
    nchunk = tpw // _CH
    wid = lax.axis_index("s") * _NC + lax.axis_index("c")
    base = wid * tpw

    def chunk_body(g, carry):
        off = base + g * _CH
        pltpu.sync_copy(vals_hbm.at[pl.ds(off, _CH)], vals_v)
        pltpu.sync_copy(ind_hbm.at[pl.ds(off, _CH)], ind_v)

        def compute(i, c2):
            v = vals_v[pl.ds(i * _LANES, _LANES)]
            ind = ind_v[pl.ds(i * _LANES, _LANES)]
            v = jnp.minimum(jnp.maximum(v, 0.0), 1.0)
            q = (v * float(_N_QUANTILES)).astype(jnp.int32) + _NUM_TOKENS
            lk = jnp.where(ind == 0, q, ind)
            lk = jnp.minimum(jnp.maximum(lk, 0), _N_QUANTILES + _NUM_TOKENS - 1)
            idx_v[pl.ds(i * _LANES, _LANES)] = lk
            return c2

        lax.fori_loop(0, _CH // _LANES, compute, 0, unroll=8)

        copies = [
            pltpu.async_copy(
                table_hbm.at[idx_v.at[pl.ds(j * _GCH, _GCH)]],
                rows_v.at[pl.ds(j * _GCH, _GCH)],
                sem,
            )
            for j in range(_CH // _GCH)
        ]
        for cp in copies:
            cp.wait()
        pltpu.sync_copy(rows_v, out_hbm.at[pl.ds(off, _CH)])
        return carry

    lax.fori_loop(0, nchunk, chunk_body, 0)


@jax.jit
def kernel(values, indicators, table):
    n, l = values.shape
    b = n * l
    vals = values.reshape(b)
    inds = indicators.reshape(b)

    run = functools.partial(
        pl.kernel,
        mesh=plsc.VectorSubcoreMesh(core_axis_name="c", subcore_axis_name="s"),
        compiler_params=pltpu.CompilerParams(use_tc_tiling_on_sc=False),
        out_type=jax.ShapeDtypeStruct((b, _D), jnp.float32),
        scratch_types=[
            pltpu.VMEM((_CH,), jnp.float32),
            pltpu.VMEM((_CH,), jnp.int32),
            pltpu.VMEM((_CH,), jnp.int32),
            pltpu.VMEM((_CH, _D), jnp.float32),
            pltpu.SemaphoreType.DMA,
        ],
    )(_embed_body)

    out = run(vals, inds, table)
    return out.reshape(n, l, _D)


# speedup vs baseline: 4.9529x; 1.1293x over previous
import functools

import jax
import jax.numpy as jnp
from jax import lax
from jax.experimental import pallas as pl
from jax.experimental.pallas import tpu as pltpu
from jax.experimental.pallas import tpu_sc as plsc

_N_QUANTILES = 1000
_NUM_TOKENS = 3
_D = 64

_NC = 2
_NS = 16
_NW = _NC * _NS
_LANES = 16

_CH = 1024
_GCH = 128


_TROWS = 1024


def _embed_body(vals_hbm, ind_hbm, table_hbm, out_hbm,
                vals_v, ind_v, idx_v, rows_v, table_sh, sem):
    b = vals_hbm.shape[0]
    tpw = b // _NW
    nchunk = tpw // _CH
    cid = lax.axis_index("c")
    sid = lax.axis_index("s")
    wid = sid * _NC + cid
    base = wid * tpw

    srow = sid * (_TROWS // _NS)
    pltpu.sync_copy(table_hbm.at[pl.ds(srow, _TROWS // _NS)],
                    rows_v.at[pl.ds(0, _TROWS // _NS)])
    pltpu.sync_copy(rows_v.at[pl.ds(0, _TROWS // _NS)],
                    table_sh.at[pl.ds(srow, _TROWS // _NS)])
    plsc.subcore_barrier()

    def chunk_body(g, carry):
        off = base + g * _CH
        pltpu.sync_copy(vals_hbm.at[pl.ds(off, _CH)], vals_v)
        pltpu.sync_copy(ind_hbm.at[pl.ds(off, _CH)], ind_v)

        def compute(i, c2):
            v = vals_v[pl.ds(i * _LANES, _LANES)]
            ind = ind_v[pl.ds(i * _LANES, _LANES)]
            v = jnp.minimum(jnp.maximum(v, 0.0), 1.0)
            q = (v * float(_N_QUANTILES)).astype(jnp.int32) + _NUM_TOKENS
            lk = jnp.where(ind == 0, q, ind)
            lk = jnp.minimum(jnp.maximum(lk, 0), _N_QUANTILES + _NUM_TOKENS - 1)
            idx_v[pl.ds(i * _LANES, _LANES)] = lk
            return c2

        lax.fori_loop(0, _CH // _LANES, compute, 0, unroll=8)

        copies = [
            pltpu.async_copy(
                table_sh.at[idx_v.at[pl.ds(j * _GCH, _GCH)]],
                rows_v.at[pl.ds(j * _GCH, _GCH)],
                sem,
            )
            for j in range(_CH // _GCH)
        ]
        for cp in copies:
            cp.wait()
        pltpu.sync_copy(rows_v, out_hbm.at[pl.ds(off, _CH)])
        return carry

    lax.fori_loop(0, nchunk, chunk_body, 0)


@jax.jit
def kernel(values, indicators, table):
    n, l = values.shape
    b = n * l
    vals = values.reshape(b)
    inds = indicators.reshape(b)

    run = functools.partial(
        pl.kernel,
        mesh=plsc.VectorSubcoreMesh(core_axis_name="c", subcore_axis_name="s"),
        compiler_params=pltpu.CompilerParams(use_tc_tiling_on_sc=False),
        out_type=jax.ShapeDtypeStruct((b, _D), jnp.float32),
        scratch_types=[
            pltpu.VMEM((_CH,), jnp.float32),
            pltpu.VMEM((_CH,), jnp.int32),
            pltpu.VMEM((_CH,), jnp.int32),
            pltpu.VMEM((_CH, _D), jnp.float32),
            pltpu.VMEM_SHARED((_TROWS, _D), jnp.float32),
            pltpu.SemaphoreType.DMA,
        ],
    )(_embed_body)

    table_p = jnp.pad(table, ((0, _TROWS - table.shape[0]), (0, 0)))
    out = run(vals, inds, table_p)
    return out.reshape(n, l, _D)


# speedup vs baseline: 5.6354x; 1.1378x over previous
import functools

import jax
import jax.numpy as jnp
from jax import lax
from jax.experimental import pallas as pl
from jax.experimental.pallas import tpu as pltpu
from jax.experimental.pallas import tpu_sc as plsc

_N_QUANTILES = 1000
_NUM_TOKENS = 3
_D = 64

_NC = 2
_NS = 16
_NW = _NC * _NS
_LANES = 16

_CH = 512
_GCH = 128
_NG = _CH // _GCH
_TROWS = 1024


def _embed_body(vals_hbm, ind_hbm, table_hbm, out_hbm,
                vals_v, ind_v, idx_v, rows_v, table_sh,
                lsem0, lsem1, gsem0, gsem1, ssem0, ssem1):
    b = vals_hbm.shape[0]
    tpw = b // _NW
    nchunk = tpw // _CH
    nstep = nchunk // 2
    cid = lax.axis_index("c")
    sid = lax.axis_index("s")
    wid = sid * _NC + cid
    base = wid * tpw

    lsem = (lsem0, lsem1)
    gsem = (gsem0, gsem1)
    ssem = (ssem0, ssem1)

    srow = sid * (_TROWS // _NS)
    pltpu.sync_copy(table_hbm.at[pl.ds(srow, _TROWS // _NS)],
                    rows_v.at[0, pl.ds(0, _TROWS // _NS)])
    pltpu.sync_copy(rows_v.at[0, pl.ds(0, _TROWS // _NS)],
                    table_sh.at[pl.ds(srow, _TROWS // _NS)])
    plsc.subcore_barrier()

    def fire_loads(g, p):
        off = base + g * _CH
        pltpu.async_copy(vals_hbm.at[pl.ds(off, _CH)], vals_v.at[p], lsem[p])
        pltpu.async_copy(ind_hbm.at[pl.ds(off, _CH)], ind_v.at[p], lsem[p])

    def wait_loads(g, p):
        off = base + g * _CH
        pltpu.make_async_copy(vals_hbm.at[pl.ds(off, _CH)], vals_v.at[p],
                              lsem[p]).wait()
        pltpu.make_async_copy(ind_hbm.at[pl.ds(off, _CH)], ind_v.at[p],
                              lsem[p]).wait()

    fire_loads(0, 0)
    fire_loads(1, 1)

    def step(t, carry):
        for p in range(2):
            g = 2 * t + p
            wait_loads(g, p)

            def compute(i, c2):
                v = vals_v[p, pl.ds(i * _LANES, _LANES)]
                ind = ind_v[p, pl.ds(i * _LANES, _LANES)]
                v = jnp.minimum(jnp.maximum(v, 0.0), 1.0)
                q = (v * float(_N_QUANTILES)).astype(jnp.int32) + _NUM_TOKENS
                lk = jnp.where(ind == 0, q, ind)
                lk = jnp.minimum(jnp.maximum(lk, 0),
                                 _N_QUANTILES + _NUM_TOKENS - 1)
                idx_v[p, pl.ds(i * _LANES, _LANES)] = lk
                return c2

            lax.fori_loop(0, _CH // _LANES, compute, 0, unroll=8)

            @pl.when(t >= 1)
            def _():
                off_prev = base + (g - 2) * _CH
                pltpu.make_async_copy(
                    rows_v.at[p], out_hbm.at[pl.ds(off_prev, _CH)],
                    ssem[p]).wait()

            copies = [
                pltpu.async_copy(
                    table_sh.at[idx_v.at[p, pl.ds(j * _GCH, _GCH)]],
                    rows_v.at[p, pl.ds(j * _GCH, _GCH)],
                    gsem[p],
                )
                for j in range(_NG)
            ]

            @pl.when(t < nstep - 1)
            def _():
                fire_loads(g + 2, p)

            for cp in copies:
                cp.wait()

            off = base + g * _CH
            pltpu.async_copy(rows_v.at[p], out_hbm.at[pl.ds(off, _CH)],
                             ssem[p])
        return carry

    lax.fori_loop(0, nstep, step, 0)

    for p in range(2):
        off = base + (nchunk - 2 + p) * _CH
        pltpu.make_async_copy(rows_v.at[p], out_hbm.at[pl.ds(off, _CH)],
                              ssem[p]).wait()


@jax.jit
def kernel(values, indicators, table):
    n, l = values.shape
    b = n * l
    vals = values.reshape(b)
    inds = indicators.reshape(b)

    run = functools.partial(
        pl.kernel,
        mesh=plsc.VectorSubcoreMesh(core_axis_name="c", subcore_axis_name="s"),
        compiler_params=pltpu.CompilerParams(use_tc_tiling_on_sc=False),
        out_type=jax.ShapeDtypeStruct((b, _D), jnp.float32),
        scratch_types=[
            pltpu.VMEM((2, _CH), jnp.float32),
            pltpu.VMEM((2, _CH), jnp.int32),
            pltpu.VMEM((2, _CH), jnp.int32),
            pltpu.VMEM((2, _CH, _D), jnp.float32),
            pltpu.VMEM_SHARED((_TROWS, _D), jnp.float32),
            pltpu.SemaphoreType.DMA,
            pltpu.SemaphoreType.DMA,
            pltpu.SemaphoreType.DMA,
            pltpu.SemaphoreType.DMA,
            pltpu.SemaphoreType.DMA,
            pltpu.SemaphoreType.DMA,
        ],
    )(_embed_body)

    table_p = jnp.pad(table, ((0, _TROWS - table.shape[0]), (0, 0)))
    out = run(vals, inds, table_p)
    return out.reshape(n, l, _D)


# speedup vs baseline: 6.9618x; 1.2354x over previous
import functools

import jax
import jax.numpy as jnp
from jax import lax
from jax.experimental import pallas as pl
from jax.experimental.pallas import tpu as pltpu
from jax.experimental.pallas import tpu_sc as plsc

_N_QUANTILES = 1000
_NUM_TOKENS = 3
_N = 16384
_L = 200
_D = 64

_NC = 2
_NS = 16
_NW = _NC * _NS
_LANES = 16

_LT = _L // 8
_NT = _N // 128
_NTW = _NT // _NW
_BLK = _NTW * 8 * 128
_TROWS = 1024


def _embed_body(v5, i5, tabt_hbm, out5,
                vblk, iblk, outb, tab_v, osem0, osem1):
    cid = lax.axis_index("c")
    sid = lax.axis_index("s")
    wid = sid * _NC + cid
    coloff = wid * _NTW * 1024

    pltpu.sync_copy(tabt_hbm, tab_v)

    osem = (osem0, osem1)

    def lt_body(lt, carry):
        pltpu.sync_copy(v5.at[lt, pl.ds(coloff, _BLK)], vblk)
        pltpu.sync_copy(i5.at[lt, pl.ds(coloff, _BLK)], iblk)

        def compute(m, c2):
            v = vblk[pl.ds(m * _LANES, _LANES)]
            ind = iblk[pl.ds(m * _LANES, _LANES)]
            v = jnp.minimum(jnp.maximum(v, 0.0), 1.0)
            q = (v * float(_N_QUANTILES)).astype(jnp.int32) + _NUM_TOKENS
            lk = jnp.where(ind == 0, q, ind)
            lk = jnp.minimum(jnp.maximum(lk, 0),
                             _N_QUANTILES + _NUM_TOKENS - 1)
            iblk[pl.ds(m * _LANES, _LANES)] = lk
            return c2

        lax.fori_loop(0, _BLK // _LANES, compute, 0, unroll=4)

        def ls_body(ls, c3):
            l = lt * 8 + ls

            def dt3_body(dt3, c4):
                for e in range(2):
                    dt = 2 * dt3 + e
                    cnt = (lt * 8 + ls) * 8 + dt

                    @pl.when(cnt >= 2)
                    def _():
                        pltpu.make_async_copy(
                            outb.at[e],
                            out5.at[l, dt, pl.ds(wid * _NTW, _NTW)],
                            osem[e]).wait()

                    def nt4_body(nt4, c5):
                        ibase = nt4 * 1024 + ls * 128
                        cols = [iblk[pl.ds(ibase + g * _LANES, _LANES)]
                                for g in range(8)]

                        def ds_body(ds, c6):
                            dbase = (dt * 8 + ds) * _TROWS
                            for g in range(8):
                                x = plsc.load_gather(tab_v, [cols[g] + dbase])
                                outb[e, nt4, ds,
                                     pl.ds(g * _LANES, _LANES)] = x
                            return c6

                        lax.fori_loop(0, 8, ds_body, 0)
                        return c5

                    lax.fori_loop(0, _NTW, nt4_body, 0)
                    pltpu.async_copy(outb.at[e],
                                     out5.at[l, dt, pl.ds(wid * _NTW, _NTW)],
                                     osem[e])
                return c4

            lax.fori_loop(0, 4, dt3_body, 0)
            return c3

        lax.fori_loop(0, 8, ls_body, 0)
        return carry

    lax.fori_loop(0, _LT, lt_body, 0)

    for e in range(2):
        pltpu.make_async_copy(outb.at[e],
                              out5.at[_L - 1, 6 + e, pl.ds(wid * _NTW, _NTW)],
                              osem[e]).wait()


@jax.jit
def kernel(values, indicators, table):
    n, l = values.shape
    v5 = (values.reshape(_NT, 128, _LT, 8).transpose(2, 0, 3, 1)
          .reshape(_LT, _NT * 8 * 128))
    i5 = (indicators.reshape(_NT, 128, _LT, 8).transpose(2, 0, 3, 1)
          .reshape(_LT, _NT * 8 * 128))
    tabt = jnp.pad(table, ((0, _TROWS - table.shape[0]), (0, 0))).T.reshape(-1)

    run = functools.partial(
        pl.kernel,
        mesh=plsc.VectorSubcoreMesh(core_axis_name="c", subcore_axis_name="s"),
        compiler_params=pltpu.CompilerParams(use_tc_tiling_on_sc=False,
                                             needs_layout_passes=False),
        out_type=jax.ShapeDtypeStruct((_L, _D // 8, _NT, 8, 128),
                                      jnp.float32),
        scratch_types=[
            pltpu.VMEM((_BLK,), jnp.float32),
            pltpu.VMEM((_BLK,), jnp.int32),
            pltpu.VMEM((2, _NTW, 8, 128), jnp.float32),
            pltpu.VMEM((_D * _TROWS,), jnp.float32),
            pltpu.SemaphoreType.DMA,
            pltpu.SemaphoreType.DMA,
        ],
    )(_embed_body)

    out5 = run(v5, i5, tabt)
    out = out5.transpose(2, 4, 0, 1, 3).reshape(n, l, _D)
    return out


# speedup vs baseline: 21.4596x; 3.0825x over previous
import functools

import jax
import jax.numpy as jnp
from jax import lax
from jax.experimental import pallas as pl
from jax.experimental.pallas import tpu as pltpu
from jax.experimental.pallas import tpu_sc as plsc

_N_QUANTILES = 1000
_NUM_TOKENS = 3
_N = 16384
_L = 200
_D = 64

_NC = 2
_NS = 16
_NW = _NC * _NS
_LANES = 16

_LT = _L // 8
_NT = _N // 128
_NTW = _NT // _NW
_BLK = _NTW * 8 * 128
_TROWS = 1024


def _embed_body(v5, i5, tabt_hbm, out5,
                vblk, iblk, outb, tab_v, osem0, osem1):
    cid = lax.axis_index("c")
    sid = lax.axis_index("s")
    wid = sid * _NC + cid
    coloff = wid * _NTW * 1024

    pltpu.sync_copy(tabt_hbm, tab_v)

    osem = (osem0, osem1)

    def lt_body(lt, carry):
        pltpu.sync_copy(v5.at[lt, pl.ds(coloff, _BLK)], vblk)
        pltpu.sync_copy(i5.at[lt, pl.ds(coloff, _BLK)], iblk)

        def compute(m, c2):
            v = vblk[pl.ds(m * _LANES, _LANES)]
            ind = iblk[pl.ds(m * _LANES, _LANES)]
            v = jnp.minimum(jnp.maximum(v, 0.0), 1.0)
            q = (v * float(_N_QUANTILES)).astype(jnp.int32) + _NUM_TOKENS
            lk = jnp.where(ind == 0, q, ind)
            lk = jnp.minimum(jnp.maximum(lk, 0),
                             _N_QUANTILES + _NUM_TOKENS - 1)
            iblk[pl.ds(m * _LANES, _LANES)] = lk
            return c2

        lax.fori_loop(0, _BLK // _LANES, compute, 0, unroll=4)

        def ls_body(ls, c3):
            l = lt * 8 + ls

            def dt3_body(dt3, c4):
                for e in range(2):
                    dt = 2 * dt3 + e
                    cnt = (lt * 8 + ls) * 8 + dt

                    @pl.when(cnt >= 2)
                    def _():
                        pltpu.make_async_copy(
                            outb.at[e],
                            out5.at[l, dt, pl.ds(wid * _NTW, _NTW)],
                            osem[e]).wait()

                    @plsc.parallel_loop(0, _NTW * 8, unroll=4)
                    def _(m):
                        nt4 = m // 8
                        g = m % 8
                        col = iblk[pl.ds(nt4 * 1024 + ls * 128 + g * _LANES,
                                         _LANES)]
                        for ds in range(8):
                            x = plsc.load_gather(
                                tab_v, [col + (dt * 8 + ds) * _TROWS])
                            outb[e, nt4, ds, pl.ds(g * _LANES, _LANES)] = x
                    pltpu.async_copy(outb.at[e],
                                     out5.at[l, dt, pl.ds(wid * _NTW, _NTW)],
                                     osem[e])
                return c4

            lax.fori_loop(0, 4, dt3_body, 0)
            return c3

        lax.fori_loop(0, 8, ls_body, 0)
        return carry

    lax.fori_loop(0, _LT, lt_body, 0)

    for e in range(2):
        pltpu.make_async_copy(outb.at[e],
                              out5.at[_L - 1, 6 + e, pl.ds(wid * _NTW, _NTW)],
                              osem[e]).wait()


@jax.jit
def kernel(values, indicators, table):
    n, l = values.shape
    v5 = (values.reshape(_NT, 128, _LT, 8).transpose(2, 0, 3, 1)
          .reshape(_LT, _NT * 8 * 128))
    i5 = (indicators.reshape(_NT, 128, _LT, 8).transpose(2, 0, 3, 1)
          .reshape(_LT, _NT * 8 * 128))
    tabt = jnp.pad(table, ((0, _TROWS - table.shape[0]), (0, 0))).T.reshape(-1)

    run = functools.partial(
        pl.kernel,
        mesh=plsc.VectorSubcoreMesh(core_axis_name="c", subcore_axis_name="s"),
        compiler_params=pltpu.CompilerParams(use_tc_tiling_on_sc=False,
                                             needs_layout_passes=False),
        out_type=jax.ShapeDtypeStruct((_L, _D // 8, _NT, 8, 128),
                                      jnp.float32),
        scratch_types=[
            pltpu.VMEM((_BLK,), jnp.float32),
            pltpu.VMEM((_BLK,), jnp.int32),
            pltpu.VMEM((2, _NTW, 8, 128), jnp.float32),
            pltpu.VMEM((_D * _TROWS,), jnp.float32),
            pltpu.SemaphoreType.DMA,
            pltpu.SemaphoreType.DMA,
        ],
    )(_embed_body)

    out5 = run(v5, i5, tabt)
    out = out5.transpose(2, 4, 0, 1, 3).reshape(n, l, _D)
    return out


# speedup vs baseline: 29.5384x; 1.3765x over previous
import functools

import jax
import jax.numpy as jnp
from jax import lax
from jax.experimental import pallas as pl
from jax.experimental.pallas import tpu as pltpu
from jax.experimental.pallas import tpu_sc as plsc

_N_QUANTILES = 1000
_NUM_TOKENS = 3
_N = 16384
_L = 200
_D = 64

_NC = 2
_NS = 16
_NW = _NC * _NS
_LANES = 16

_LT = _L // 8
_NT = _N // 128
_NTW = _NT // _NW
_BLK = _NTW * 8 * 128
_TROWS = 1024


def _embed_body(v5, i5, tabt_hbm, out5,
                vblk, iblk, outb, tab_v, lsem, osem0, osem1):
    cid = lax.axis_index("c")
    sid = lax.axis_index("s")
    wid = sid * _NC + cid
    coloff = wid * _NTW * 1024

    pltpu.sync_copy(tabt_hbm, tab_v)

    osem = (osem0, osem1)

    def fire_in(lt, boff):
        pltpu.async_copy(v5.at[lt, pl.ds(coloff, _BLK)],
                         vblk.at[pl.ds(boff, _BLK)], lsem)
        pltpu.async_copy(i5.at[lt, pl.ds(coloff, _BLK)],
                         iblk.at[pl.ds(boff, _BLK)], lsem)

    fire_in(0, 0)

    def lt_body(lt, carry):
        qoff = (lt % 2) * _BLK
        pltpu.make_async_copy(v5.at[lt, pl.ds(coloff, _BLK)],
                              vblk.at[pl.ds(qoff, _BLK)], lsem).wait()
        pltpu.make_async_copy(i5.at[lt, pl.ds(coloff, _BLK)],
                              iblk.at[pl.ds(qoff, _BLK)], lsem).wait()

        @pl.when(lt + 1 < _LT)
        def _():
            fire_in(lt + 1, _BLK - qoff)

        @plsc.parallel_loop(0, _BLK // _LANES, unroll=4)
        def _(m):
            v = vblk[pl.ds(qoff + m * _LANES, _LANES)]
            ind = iblk[pl.ds(qoff + m * _LANES, _LANES)]
            v = jnp.minimum(jnp.maximum(v, 0.0), 1.0)
            q = (v * float(_N_QUANTILES)).astype(jnp.int32) + _NUM_TOKENS
            lk = jnp.where(ind == 0, q, ind)
            lk = jnp.minimum(jnp.maximum(lk, 0),
                             _N_QUANTILES + _NUM_TOKENS - 1)
            iblk[pl.ds(qoff + m * _LANES, _LANES)] = lk

        def ls_body(ls, c3):
            l = lt * 8 + ls

            def dt3_body(dt3, c4):
                for e in range(2):
                    dt = 2 * dt3 + e
                    cnt = (lt * 8 + ls) * 8 + dt

                    @pl.when(cnt >= 2)
                    def _():
                        pltpu.make_async_copy(
                            outb.at[e],
                            out5.at[l, dt, pl.ds(wid * _NTW, _NTW)],
                            osem[e]).wait()

                    @plsc.parallel_loop(0, _NTW * 8, unroll=8)
                    def _(m):
                        nt4 = m // 8
                        g = m % 8
                        col = iblk[pl.ds(qoff + nt4 * 1024 + ls * 128
                                         + g * _LANES, _LANES)]
                        for ds in range(8):
                            x = plsc.load_gather(
                                tab_v, [col + (dt * 8 + ds) * _TROWS])
                            outb[e, nt4, ds, pl.ds(g * _LANES, _LANES)] = x
                    pltpu.async_copy(outb.at[e],
                                     out5.at[l, dt, pl.ds(wid * _NTW, _NTW)],
                                     osem[e])
                return c4

            lax.fori_loop(0, 4, dt3_body, 0)
            return c3

        lax.fori_loop(0, 8, ls_body, 0)
        return carry

    lax.fori_loop(0, _LT, lt_body, 0)

    for e in range(2):
        pltpu.make_async_copy(outb.at[e],
                              out5.at[_L - 1, 6 + e, pl.ds(wid * _NTW, _NTW)],
                              osem[e]).wait()


@jax.jit
def kernel(values, indicators, table):
    n, l = values.shape
    v5 = (values.reshape(_NT, 128, _LT, 8).transpose(2, 0, 3, 1)
          .reshape(_LT, _NT * 8 * 128))
    i5 = (indicators.reshape(_NT, 128, _LT, 8).transpose(2, 0, 3, 1)
          .reshape(_LT, _NT * 8 * 128))
    tabt = jnp.pad(table, ((0, _TROWS - table.shape[0]), (0, 0))).T.reshape(-1)

    run = functools.partial(
        pl.kernel,
        mesh=plsc.VectorSubcoreMesh(core_axis_name="c", subcore_axis_name="s"),
        compiler_params=pltpu.CompilerParams(use_tc_tiling_on_sc=False,
                                             needs_layout_passes=False),
        out_type=jax.ShapeDtypeStruct((_L, _D // 8, _NT, 8, 128),
                                      jnp.float32),
        scratch_types=[
            pltpu.VMEM((2 * _BLK,), jnp.float32),
            pltpu.VMEM((2 * _BLK,), jnp.int32),
            pltpu.VMEM((2, _NTW, 8, 128), jnp.float32),
            pltpu.VMEM((_D * _TROWS,), jnp.float32),
            pltpu.SemaphoreType.DMA,
            pltpu.SemaphoreType.DMA,
            pltpu.SemaphoreType.DMA,
        ],
    )(_embed_body)

    out5 = run(v5, i5, tabt)
    out = out5.transpose(2, 4, 0, 1, 3).reshape(n, l, _D)
    return out
